# baseline (speedup 1.0000x reference)
import functools

import jax
import jax.numpy as jnp
from jax import lax
from jax.experimental import pallas as pl

ICP_ITERS = 5
HUBER_DELTA = 1.0
TRIM_DIST = 5.0
BFAR_TEMP = 10.0
CHUNK = 1024


def _icp_kernel(scanT_ref, inten_ref, map_ref, mapT_ref, T0_ref, params_ref,
                out_ref, *, n_pts, n_map):
    N = n_pts
    M = n_map
    C = CHUNK
    n_chunks = M // C

    scanTb = scanT_ref[0]
    inten = inten_ref[0]
    T = T0_ref[0]
    prm = params_ref[...]

    a = jnp.maximum(prm[0, 0], 0.0)
    b = jnp.maximum(prm[0, 1], 0.0)
    thresh = a * jnp.mean(inten) + b
    w_bfar = jax.nn.sigmoid((inten - thresh) * BFAR_TEMP)

    for _ in range(ICP_ITERS):
        Rb = T[:3, :3].astype(jnp.bfloat16)
        sT = lax.dot_general(Rb, scanTb, (((1,), (0,)), ((), ())),
                             preferred_element_type=jnp.float32)
        sx = sT[0:1, :] + T[0, 3]
        sy = sT[1:2, :] + T[1, 3]
        sz = sT[2:3, :] + T[2, 3]

        UNROLL = 8
        n_strides = M // (8 * UNROLL)

        def stripe_body(si, carry, sx=sx, sy=sy, sz=sz):
            av, ast = carry
            for u in range(UNROLL):
                st = si * UNROLL + u
                m8 = map_ref[0, pl.ds(st * 8, 8), :]
                dx = m8[:, 0:1] - sx
                dy = m8[:, 1:2] - sy
                dz = m8[:, 2:3] - sz
                d2 = dx * dx + dy * dy + dz * dz
                better = d2 < av
                av = jnp.where(better, d2, av)
                ast = jnp.where(better, st, ast)
            return av, ast

        av0 = jnp.full((8, N), jnp.inf, dtype=jnp.float32)
        ast0 = jnp.zeros((8, N), dtype=jnp.int32)
        av, ast = lax.fori_loop(0, n_strides, stripe_body, (av0, ast0))

        rowio = lax.broadcasted_iota(jnp.int32, (8, N), 0)
        idx8 = ast * 8 + rowio

        def comb(v1, i1, v2, i2):
            take2 = (v2 < v1) | ((v2 == v1) & (i2 < i1))
            return jnp.where(take2, v2, v1), jnp.where(take2, i2, i1)

        v4, i4 = comb(av[0:4, :], idx8[0:4, :], av[4:8, :], idx8[4:8, :])
        v2, i2 = comb(v4[0:2, :], i4[0:2, :], v4[2:4, :], i4[2:4, :])
        _, idx = comb(v2[0:1, :], i2[0:1, :], v2[1:2, :], i2[1:2, :])

        def gather_body(ci, nn, idx=idx):
            iota = lax.broadcasted_iota(jnp.int32, (C, N), 0) + ci * C
            onehot = (iota == idx).astype(jnp.float32)
            mT = mapT_ref[0, :, pl.ds(ci * C, C)]
            return nn + lax.dot_general(
                mT, onehot, (((1,), (0,)), ((), ())),
                preferred_element_type=jnp.float32)

        nn = lax.fori_loop(0, n_chunks, gather_body,
                           jnp.zeros((3, N), dtype=jnp.float32))

        nx = nn[0:1, :]
        ny = nn[1:2, :]
        nz = nn[2:3, :]
        rx = nx - sx
        ry = ny - sy
        rz = nz - sz
        d2r = rx * rx + ry * ry + rz * rz
        dist = jnp.sqrt(d2r + 1e-12)
        w_h = jnp.where(dist < HUBER_DELTA, 1.0, HUBER_DELTA / dist)
        w = w_bfar * w_h * (dist < TRIM_DIST).astype(jnp.float32)
        wsum = jnp.sum(w) + 1e-8

        mu_sx = jnp.sum(w * sx) / wsum
        mu_sy = jnp.sum(w * sy) / wsum
        mu_mx = jnp.sum(w * nx) / wsum
        mu_my = jnp.sum(w * ny) / wsum
        sc0 = sx - mu_sx
        sc1 = sy - mu_sy
        mc0 = nx - mu_mx
        mc1 = ny - mu_my
        cross = jnp.sum(w * (sc0 * mc1 - sc1 * mc0))
        dot = jnp.sum(w * (sc0 * mc0 + sc1 * mc1))
        h = jnp.sqrt(cross * cross + dot * dot)
        safe = h > 0.0
        c = jnp.where(safe, dot / jnp.where(safe, h, 1.0), 1.0)
        sn = jnp.where(safe, cross / jnp.where(safe, h, 1.0), 0.0)
        t2x = mu_mx - (c * mu_sx - sn * mu_sy)
        t2y = mu_my - (sn * mu_sx + c * mu_sy)

        row0 = c * T[0:1, :] - sn * T[1:2, :] + t2x * T[3:4, :]
        row1 = sn * T[0:1, :] + c * T[1:2, :] + t2y * T[3:4, :]
        T = jnp.concatenate([row0, row1, T[2:3, :], T[3:4, :]], axis=0)

    out_ref[0] = T


def kernel(scan_pc, scan_intensity, map_pc, T_init, params):
    B, N, _ = scan_pc.shape
    M = map_pc.shape[1]
    scanT = jnp.transpose(scan_pc, (0, 2, 1)).astype(jnp.bfloat16)
    mapT = jnp.transpose(map_pc, (0, 2, 1))
    inten3 = scan_intensity[:, None, :]
    prm2 = params.reshape(1, 2)

    f = functools.partial(_icp_kernel, n_pts=N, n_map=M)
    return pl.pallas_call(
        f,
        grid=(B,),
        in_specs=[
            pl.BlockSpec((1, 3, N), lambda i: (i, 0, 0)),
            pl.BlockSpec((1, 1, N), lambda i: (i, 0, 0)),
            pl.BlockSpec((1, M, 3), lambda i: (i, 0, 0)),
            pl.BlockSpec((1, 3, M), lambda i: (i, 0, 0)),
            pl.BlockSpec((1, 4, 4), lambda i: (i, 0, 0)),
            pl.BlockSpec((1, 2), lambda i: (0, 0)),
        ],
        out_specs=pl.BlockSpec((1, 4, 4), lambda i: (i, 0, 0)),
        out_shape=jax.ShapeDtypeStruct((B, 4, 4), jnp.float32),
    )(scanT, inten3, map_pc, mapT, T_init, prm2)


# speedup vs baseline: 1.2101x; 1.2101x over previous
import functools

import jax
import jax.numpy as jnp
from jax import lax
from jax.experimental import pallas as pl

ICP_ITERS = 5
HUBER_DELTA = 1.0
TRIM_DIST = 5.0
BFAR_TEMP = 10.0
CHUNK = 2048


def _icp_kernel(scanT_ref, inten_ref, map_ref, mapT_ref, T0_ref, params_ref,
                out_ref, *, n_pts, n_map):
    N = n_pts
    M = n_map
    C = CHUNK
    n_chunks = M // C

    scanTb = scanT_ref[0]
    inten = inten_ref[0]
    T = T0_ref[0]
    prm = params_ref[...]

    a = jnp.maximum(prm[0, 0], 0.0)
    b = jnp.maximum(prm[0, 1], 0.0)
    thresh = a * jnp.mean(inten) + b
    w_bfar = jax.nn.sigmoid((inten - thresh) * BFAR_TEMP)

    for _ in range(ICP_ITERS):
        Rb = T[:3, :3].astype(jnp.bfloat16)
        sT = lax.dot_general(Rb, scanTb, (((1,), (0,)), ((), ())),
                             preferred_element_type=jnp.float32)
        sx = sT[0:1, :] + T[0, 3]
        sy = sT[1:2, :] + T[1, 3]
        sz = sT[2:3, :] + T[2, 3]

        def chunk_body(ci, carry, sx=sx, sy=sy, sz=sz):
            run_min, run_arg = carry
            m = map_ref[0, pl.ds(ci * C, C), :]
            dx = m[:, 0:1] - sx
            dy = m[:, 1:2] - sy
            dz = m[:, 2:3] - sz
            d2 = dx * dx + dy * dy + dz * dz
            tmin = jnp.min(d2, axis=0, keepdims=True)
            iota = lax.broadcasted_iota(jnp.int32, (C, N), 0) + ci * C
            targ = jnp.min(jnp.where(d2 == tmin, iota, M),
                           axis=0, keepdims=True)
            better = tmin < run_min
            return (jnp.where(better, tmin, run_min),
                    jnp.where(better, targ, run_arg))

        run_min0 = jnp.full((1, N), jnp.inf, dtype=jnp.float32)
        run_arg0 = jnp.zeros((1, N), dtype=jnp.int32)
        _, idx = lax.fori_loop(0, n_chunks, chunk_body, (run_min0, run_arg0))

        def gather_body(ci, nn, idx=idx):
            iota = lax.broadcasted_iota(jnp.int32, (C, N), 0) + ci * C
            onehot = (iota == idx).astype(jnp.float32)
            mT = mapT_ref[0, :, pl.ds(ci * C, C)]
            return nn + lax.dot_general(
                mT, onehot, (((1,), (0,)), ((), ())),
                preferred_element_type=jnp.float32)

        nn = lax.fori_loop(0, n_chunks, gather_body,
                           jnp.zeros((3, N), dtype=jnp.float32))

        nx = nn[0:1, :]
        ny = nn[1:2, :]
        nz = nn[2:3, :]
        rx = nx - sx
        ry = ny - sy
        rz = nz - sz
        d2r = rx * rx + ry * ry + rz * rz
        dist = jnp.sqrt(d2r + 1e-12)
        w_h = jnp.where(dist < HUBER_DELTA, 1.0, HUBER_DELTA / dist)
        w = w_bfar * w_h * (dist < TRIM_DIST).astype(jnp.float32)
        wsum = jnp.sum(w) + 1e-8

        mu_sx = jnp.sum(w * sx) / wsum
        mu_sy = jnp.sum(w * sy) / wsum
        mu_mx = jnp.sum(w * nx) / wsum
        mu_my = jnp.sum(w * ny) / wsum
        sc0 = sx - mu_sx
        sc1 = sy - mu_sy
        mc0 = nx - mu_mx
        mc1 = ny - mu_my
        cross = jnp.sum(w * (sc0 * mc1 - sc1 * mc0))
        dot = jnp.sum(w * (sc0 * mc0 + sc1 * mc1))
        h = jnp.sqrt(cross * cross + dot * dot)
        safe = h > 0.0
        c = jnp.where(safe, dot / jnp.where(safe, h, 1.0), 1.0)
        sn = jnp.where(safe, cross / jnp.where(safe, h, 1.0), 0.0)
        t2x = mu_mx - (c * mu_sx - sn * mu_sy)
        t2y = mu_my - (sn * mu_sx + c * mu_sy)

        row0 = c * T[0:1, :] - sn * T[1:2, :] + t2x * T[3:4, :]
        row1 = sn * T[0:1, :] + c * T[1:2, :] + t2y * T[3:4, :]
        T = jnp.concatenate([row0, row1, T[2:3, :], T[3:4, :]], axis=0)

    out_ref[0] = T


def kernel(scan_pc, scan_intensity, map_pc, T_init, params):
    B, N, _ = scan_pc.shape
    M = map_pc.shape[1]
    scanT = jnp.transpose(scan_pc, (0, 2, 1)).astype(jnp.bfloat16)
    mapT = jnp.transpose(map_pc, (0, 2, 1))
    inten3 = scan_intensity[:, None, :]
    prm2 = params.reshape(1, 2)

    f = functools.partial(_icp_kernel, n_pts=N, n_map=M)
    return pl.pallas_call(
        f,
        grid=(B,),
        in_specs=[
            pl.BlockSpec((1, 3, N), lambda i: (i, 0, 0)),
            pl.BlockSpec((1, 1, N), lambda i: (i, 0, 0)),
            pl.BlockSpec((1, M, 3), lambda i: (i, 0, 0)),
            pl.BlockSpec((1, 3, M), lambda i: (i, 0, 0)),
            pl.BlockSpec((1, 4, 4), lambda i: (i, 0, 0)),
            pl.BlockSpec((1, 2), lambda i: (0, 0)),
        ],
        out_specs=pl.BlockSpec((1, 4, 4), lambda i: (i, 0, 0)),
        out_shape=jax.ShapeDtypeStruct((B, 4, 4), jnp.float32),
    )(scanT, inten3, map_pc, mapT, T_init, prm2)
